# Initial kernel scaffold; baseline (speedup 1.0000x reference)
import jax
import jax.numpy as jnp
from jax.experimental import pallas as pl


def kernel(x, edge_index, edge_attr, u, params):
    raise NotImplementedError("write your pallas kernel here")


# speedup vs baseline: 2.5373x; 2.5373x over previous
import functools

import jax
import jax.numpy as jnp
from jax import lax
from jax.experimental import pallas as pl
from jax.experimental.pallas import tpu as pltpu
from jax.experimental.pallas import tpu_sc as plsc

H = 16
_EDGE_BLOCK = 8000


def _ln_relu(h, g, bt):
    mu = jnp.mean(h, axis=-1, keepdims=True)
    var = jnp.mean((h - mu) ** 2, axis=-1, keepdims=True)
    return jax.nn.relu((h - mu) * lax.rsqrt(var + 1e-5) * g + bt)


def _node_encode_body(x_ref, u_ref, wn_ref, bn_ref, gn_ref, btn_ref,
                      wg_ref, bg_ref, gg_ref, btg_ref,
                      w2_ref, w3_ref, w4_ref, bce_ref,
                      n_ref, p2_ref, p3_ref, c_ref):
    n = _ln_relu(x_ref[...] @ wn_ref[...] + bn_ref[...], gn_ref[...], btn_ref[...])
    n_ref[...] = n
    p2_ref[...] = n @ w2_ref[...]
    p3_ref[...] = n @ w3_ref[...]
    g = _ln_relu(u_ref[...] @ wg_ref[...] + bg_ref[...], gg_ref[...], btg_ref[...])
    c_ref[...] = g @ w4_ref[...] + bce_ref[...]


def _node_encode(x, u, wn, bn, gn, btn, wg, bg, gg, btg, w2, w3, w4, bce,
                 interpret=False):
    n_nodes = x.shape[0]
    out_shape = [
        jax.ShapeDtypeStruct((n_nodes, H), jnp.float32),
        jax.ShapeDtypeStruct((n_nodes, H), jnp.float32),
        jax.ShapeDtypeStruct((n_nodes, H), jnp.float32),
        jax.ShapeDtypeStruct((1, H), jnp.float32),
    ]
    return pl.pallas_call(_node_encode_body, out_shape=out_shape,
                          interpret=interpret)(
        x, u, wn, bn, gn, btn, wg, bg, gg, btg, w2, w3, w4, bce)


@functools.cache
def _make_gather(n_edges):
    nw = 32
    per = n_edges // nw
    ch = 2000
    nch = per // ch
    mesh = plsc.VectorSubcoreMesh(core_axis_name="c", subcore_axis_name="s")

    @functools.partial(
        pl.kernel, mesh=mesh,
        compiler_params=pltpu.CompilerParams(use_tc_tiling_on_sc=False),
        out_type=[jax.ShapeDtypeStruct((n_edges, H), jnp.float32),
                  jax.ShapeDtypeStruct((n_edges, H), jnp.float32)],
        scratch_types=[pltpu.VMEM((ch,), jnp.int32),
                       pltpu.VMEM((ch,), jnp.int32),
                       pltpu.VMEM((ch, H), jnp.float32),
                       pltpu.VMEM((ch, H), jnp.float32),
                       pltpu.SemaphoreType.DMA,
                       pltpu.SemaphoreType.DMA],
    )
    def gather(p2_hbm, p3_hbm, src_hbm, dst_hbm, gs_hbm, gd_hbm,
               idxa, idxb, rowsa, rowsb, sema, semb):
        wid = lax.axis_index("s") * 2 + lax.axis_index("c")
        base = wid * per

        def body(c, carry):
            off = base + c * ch
            pltpu.sync_copy(src_hbm.at[pl.ds(off, ch)], idxa)
            pltpu.sync_copy(dst_hbm.at[pl.ds(off, ch)], idxb)
            cpa = pltpu.async_copy(p2_hbm.at[idxa], rowsa, sema)
            cpb = pltpu.async_copy(p3_hbm.at[idxb], rowsb, semb)
            cpa.wait()
            pltpu.sync_copy(rowsa, gs_hbm.at[pl.ds(off, ch)])
            cpb.wait()
            pltpu.sync_copy(rowsb, gd_hbm.at[pl.ds(off, ch)])
            return carry

        lax.fori_loop(0, nch, body, 0)

    return gather


@functools.cache
def _make_scatter(n_edges, n_nodes):
    nw = 32
    per = n_edges // nw
    ch = 2000
    nch = per // ch
    zrows = n_nodes // 16
    mesh = plsc.VectorSubcoreMesh(core_axis_name="c", subcore_axis_name="s")

    @functools.partial(
        pl.kernel, mesh=mesh,
        compiler_params=pltpu.CompilerParams(use_tc_tiling_on_sc=False),
        out_type=jax.ShapeDtypeStruct((2, n_nodes, H), jnp.float32),
        scratch_types=[pltpu.VMEM((ch,), jnp.int32),
                       pltpu.VMEM((ch, H), jnp.float32),
                       pltpu.VMEM((zrows, H), jnp.float32),
                       pltpu.VMEM_SHARED((n_nodes, H), jnp.float32),
                       pltpu.SemaphoreType.DMA],
    )
    def scatter(e2_hbm, dst_hbm, out_hbm, idx, rows, tbuf, acc_sh, sem):
        cid = lax.axis_index("c")
        sid = lax.axis_index("s")

        def zero_row(r, carry):
            tbuf[r, :] = jnp.zeros((H,), jnp.float32)
            return carry

        lax.fori_loop(0, zrows, zero_row, 0)
        pltpu.sync_copy(tbuf, acc_sh.at[pl.ds(sid * zrows, zrows)])
        plsc.subcore_barrier()

        wid = sid * 2 + cid
        base = wid * per

        def body(c, carry):
            off = base + c * ch
            pltpu.sync_copy(dst_hbm.at[pl.ds(off, ch)], idx)
            cp = pltpu.async_copy(e2_hbm.at[pl.ds(off, ch)], rows, sem)
            cp.wait()
            pltpu.sync_copy(rows, acc_sh.at[idx], add=True)
            return carry

        lax.fori_loop(0, nch, body, 0)
        plsc.subcore_barrier()
        pltpu.sync_copy(acc_sh.at[pl.ds(sid * zrows, zrows)], tbuf)
        pltpu.sync_copy(tbuf, out_hbm.at[cid].at[pl.ds(sid * zrows, zrows)])

    return scatter


def _edge_core_body(ea_ref, gs_ref, gd_ref, c_ref,
                    we_ref, be_ref, ge_ref, bte_ref,
                    w1_ref, gc_ref, btc_ref, woe_ref, boe_ref,
                    e2_ref, ze_ref, m_ref, s_ref):
    i = pl.program_id(0)
    e_enc = _ln_relu(ea_ref[...] @ we_ref[...] + be_ref[...],
                     ge_ref[...], bte_ref[...])
    h = e_enc @ w1_ref[...] + gs_ref[...] + gd_ref[...] + c_ref[...]
    e2 = _ln_relu(h, gc_ref[...], btc_ref[...])
    e2_ref[...] = e2
    ze = e2 @ woe_ref[...] + boe_ref[...]
    ze_ref[...] = ze

    @pl.when(i == 0)
    def _():
        m_ref[...] = jnp.full((1, 2), -jnp.inf, jnp.float32)
        s_ref[...] = jnp.zeros((1, 2), jnp.float32)

    bm = jnp.max(ze, axis=0, keepdims=True)
    m_old = m_ref[...]
    m_new = jnp.maximum(m_old, bm)
    s_ref[...] = (s_ref[...] * jnp.exp(m_old - m_new)
                  + jnp.sum(jnp.exp(ze - m_new), axis=0, keepdims=True))
    m_ref[...] = m_new


def _edge_core(ea, gs, gd, c, we, be, ge, bte, w1, gc, btc, woe, boe,
               interpret=False):
    n_edges = ea.shape[0]
    blk = _EDGE_BLOCK
    grid = n_edges // blk
    row_spec = pl.BlockSpec((blk, H), lambda i: (i, 0))
    full = lambda a: pl.BlockSpec(a.shape, lambda i: tuple(0 for _ in a.shape))
    out_shape = [
        jax.ShapeDtypeStruct((n_edges, H), jnp.float32),
        jax.ShapeDtypeStruct((n_edges, 2), jnp.float32),
        jax.ShapeDtypeStruct((1, 2), jnp.float32),
        jax.ShapeDtypeStruct((1, 2), jnp.float32),
    ]
    return pl.pallas_call(
        _edge_core_body,
        grid=(grid,),
        in_specs=[row_spec, row_spec, row_spec, full(c), full(we), full(be),
                  full(ge), full(bte), full(w1), full(gc), full(btc),
                  full(woe), full(boe)],
        out_specs=[row_spec, pl.BlockSpec((blk, 2), lambda i: (i, 0)),
                   pl.BlockSpec((1, 2), lambda i: (0, 0)),
                   pl.BlockSpec((1, 2), lambda i: (0, 0))],
        out_shape=out_shape,
        interpret=interpret,
    )(ea, gs, gd, c, we, be, ge, bte, w1, gc, btc, woe, boe)


def _edge_norm_body(ze_ref, m_ref, s_ref, out_ref):
    out_ref[...] = jnp.exp(ze_ref[...] - m_ref[...]) / s_ref[...]


def _edge_norm(ze, m, s, interpret=False):
    n_edges = ze.shape[0]
    blk = _EDGE_BLOCK
    grid = n_edges // blk
    spec2 = pl.BlockSpec((blk, 2), lambda i: (i, 0))
    stat = pl.BlockSpec((1, 2), lambda i: (0, 0))
    return pl.pallas_call(
        _edge_norm_body,
        grid=(grid,),
        in_specs=[spec2, stat, stat],
        out_specs=spec2,
        out_shape=jax.ShapeDtypeStruct((n_edges, 2), jnp.float32),
        interpret=interpret,
    )(ze, m, s)


def _node_core_body(n_ref, agg_ref, wnn_ref, wna_ref, bcn_ref, gcn_ref,
                    btcn_ref, won_ref, bon_ref, out_ref):
    agg = agg_ref[0] + agg_ref[1]
    h = n_ref[...] @ wnn_ref[...] + agg @ wna_ref[...] + bcn_ref[...]
    n2 = _ln_relu(h, gcn_ref[...], btcn_ref[...])
    zn = n2 @ won_ref[...] + bon_ref[...]
    m = jnp.max(zn, axis=0, keepdims=True)
    ez = jnp.exp(zn - m)
    out_ref[...] = ez / jnp.sum(ez, axis=0, keepdims=True)


def _node_core(n, aggp, wnn, wna, bcn, gcn, btcn, won, bon, interpret=False):
    n_nodes = n.shape[0]
    return pl.pallas_call(
        _node_core_body,
        out_shape=jax.ShapeDtypeStruct((n_nodes, 2), jnp.float32),
        interpret=interpret,
    )(n, aggp, wnn, wna, bcn, gcn, btcn, won, bon)


def kernel(x, edge_index, edge_attr, u, params):
    src = edge_index[0].astype(jnp.int32)
    dst = edge_index[1].astype(jnp.int32)
    r = lambda v: jnp.reshape(v, (1, -1))

    pe, pn, pg = params["enc_e"], params["enc_n"], params["enc_g"]
    ce, cn = params["core_e"], params["core_n"]
    oe, on = params["out_e"], params["out_n"]
    w1, w2, w3, w4 = (ce["W"][0:16], ce["W"][16:32], ce["W"][32:48],
                      ce["W"][48:64])
    wnn, wna = cn["W"][0:16], cn["W"][16:32]

    n, p2, p3, c = _node_encode(
        x, u, pn["W"], r(pn["b"]), r(pn["g"]), r(pn["bt"]),
        pg["W"], r(pg["b"]), r(pg["g"]), r(pg["bt"]),
        w2, w3, w4, r(ce["b"]))

    gs, gd = _make_gather(edge_attr.shape[0])(p2, p3, src, dst)

    e2, ze, m, s = _edge_core(
        edge_attr, gs, gd, c, pe["W"], r(pe["b"]), r(pe["g"]), r(pe["bt"]),
        w1, r(ce["g"]), r(ce["bt"]), oe["W"], r(oe["b"]))

    edge_out = _edge_norm(ze, m, s)

    aggp = _make_scatter(edge_attr.shape[0], x.shape[0])(e2, dst)

    node_out = _node_core(n, aggp, wnn, wna, r(cn["b"]), r(cn["g"]),
                          r(cn["bt"]), on["W"], r(on["b"]))

    glob_out = jnp.ones((1, 1), jnp.float32)
    return edge_out, node_out, glob_out


# speedup vs baseline: 4.7428x; 1.8693x over previous
import functools

import jax
import jax.numpy as jnp
from jax import lax
from jax.experimental import pallas as pl
from jax.experimental.pallas import tpu as pltpu
from jax.experimental.pallas import tpu_sc as plsc

H = 16
_PACK = 8
_LANES = _PACK * H
_EDGE_PBLOCK = 5000


def _ln_relu_packed(h, m_ref, g_t, bt_t):
    mavg = m_ref[...]
    mu = h @ mavg
    d = h - mu
    var = (d * d) @ mavg
    return jax.nn.relu(d * lax.rsqrt(var + 1e-5) * g_t + bt_t)


def _node_encode_body(xr_ref, u_ref, wnbd_ref, bn_ref, gn_ref, btn_ref,
                      mavg_ref, w2bd_ref, w3bd_ref,
                      wg_ref, bg_ref, gg_ref, btg_ref, w4t_ref, bce_ref,
                      n_ref, p2_ref, p3_ref, c_ref):
    h = xr_ref[...] @ wnbd_ref[...] + bn_ref[...]
    n = _ln_relu_packed(h, mavg_ref, gn_ref[...], btn_ref[...])
    n_ref[...] = n
    p2_ref[...] = n @ w2bd_ref[...]
    p3_ref[...] = n @ w3bd_ref[...]
    hg = u_ref[...] @ wg_ref[...] + bg_ref[...]
    mu = jnp.mean(hg, axis=-1, keepdims=True)
    var = jnp.mean((hg - mu) ** 2, axis=-1, keepdims=True)
    g = jax.nn.relu((hg - mu) * lax.rsqrt(var + 1e-5) * gg_ref[...]
                    + btg_ref[...])
    c_ref[...] = g @ w4t_ref[...] + bce_ref[...]


def _node_encode(xr, u, wnbd, bn, gn, btn, mavg, w2bd, w3bd,
                 wg, bg, gg, btg, w4t, bce, interpret=False):
    rows = xr.shape[0]
    out_shape = [
        jax.ShapeDtypeStruct((rows, _LANES), jnp.float32),
        jax.ShapeDtypeStruct((rows, _LANES), jnp.float32),
        jax.ShapeDtypeStruct((rows, _LANES), jnp.float32),
        jax.ShapeDtypeStruct((1, _LANES), jnp.float32),
    ]
    return pl.pallas_call(_node_encode_body, out_shape=out_shape,
                          interpret=interpret)(
        xr, u, wnbd, bn, gn, btn, mavg, w2bd, w3bd,
        wg, bg, gg, btg, w4t, bce)


@functools.cache
def _make_gather(n_edges):
    nw = 32
    per = n_edges // nw
    ch = 2000
    nch = per // ch
    mesh = plsc.VectorSubcoreMesh(core_axis_name="c", subcore_axis_name="s")

    @functools.partial(
        pl.kernel, mesh=mesh,
        compiler_params=pltpu.CompilerParams(use_tc_tiling_on_sc=False),
        out_type=[jax.ShapeDtypeStruct((n_edges, H), jnp.float32),
                  jax.ShapeDtypeStruct((n_edges, H), jnp.float32),
                  jax.ShapeDtypeStruct((n_edges * H,), jnp.float32)],
        scratch_types=[pltpu.VMEM((ch,), jnp.int32),
                       pltpu.VMEM((ch,), jnp.int32),
                       pltpu.VMEM((ch, H), jnp.float32),
                       pltpu.VMEM((ch, H), jnp.float32),
                       pltpu.VMEM((ch * H,), jnp.float32),
                       pltpu.SemaphoreType.DMA,
                       pltpu.SemaphoreType.DMA,
                       pltpu.SemaphoreType.DMA],
    )
    def gather(p2_hbm, p3_hbm, src_hbm, dst_hbm, ea_hbm,
               gs_hbm, gd_hbm, ea_out_hbm,
               idxa, idxb, rowsa, rowsb, rowse, sema, semb, seme):
        wid = lax.axis_index("s") * 2 + lax.axis_index("c")
        base = wid * per

        def body(c, carry):
            off = base + c * ch
            pltpu.sync_copy(src_hbm.at[pl.ds(off, ch)], idxa)
            pltpu.sync_copy(dst_hbm.at[pl.ds(off, ch)], idxb)
            cpa = pltpu.async_copy(p2_hbm.at[idxa], rowsa, sema)
            cpb = pltpu.async_copy(p3_hbm.at[idxb], rowsb, semb)
            cpe = pltpu.async_copy(ea_hbm.at[pl.ds(off * H, ch * H)],
                                   rowse, seme)
            cpa.wait()
            pltpu.sync_copy(rowsa, gs_hbm.at[pl.ds(off, ch)])
            cpb.wait()
            pltpu.sync_copy(rowsb, gd_hbm.at[pl.ds(off, ch)])
            cpe.wait()
            pltpu.sync_copy(rowse, ea_out_hbm.at[pl.ds(off * H, ch * H)])
            return carry

        lax.fori_loop(0, nch, body, 0)

    return gather


@functools.cache
def _make_scatter(n_edges, n_nodes):
    nw = 32
    per = n_edges // nw
    ch = 2000
    nch = per // ch
    zrows = n_nodes // 16
    mesh = plsc.VectorSubcoreMesh(core_axis_name="c", subcore_axis_name="s")

    @functools.partial(
        pl.kernel, mesh=mesh,
        compiler_params=pltpu.CompilerParams(use_tc_tiling_on_sc=False),
        out_type=jax.ShapeDtypeStruct((2, n_nodes, H), jnp.float32),
        scratch_types=[pltpu.VMEM((ch,), jnp.int32),
                       pltpu.VMEM((ch, H), jnp.float32),
                       pltpu.VMEM((zrows, H), jnp.float32),
                       pltpu.VMEM_SHARED((n_nodes, H), jnp.float32),
                       pltpu.SemaphoreType.DMA],
    )
    def scatter(e2_hbm, dst_hbm, out_hbm, idx, rows, tbuf, acc_sh, sem):
        cid = lax.axis_index("c")
        sid = lax.axis_index("s")

        def zero_row(r, carry):
            tbuf[r, :] = jnp.zeros((H,), jnp.float32)
            return carry

        lax.fori_loop(0, zrows, zero_row, 0)
        pltpu.sync_copy(tbuf, acc_sh.at[pl.ds(sid * zrows, zrows)])
        plsc.subcore_barrier()

        wid = sid * 2 + cid
        base = wid * per

        def body(c, carry):
            off = base + c * ch
            pltpu.sync_copy(dst_hbm.at[pl.ds(off, ch)], idx)
            cp = pltpu.async_copy(e2_hbm.at[pl.ds(off, ch)], rows, sem)
            cp.wait()
            pltpu.sync_copy(rows, acc_sh.at[idx], add=True)
            return carry

        lax.fori_loop(0, nch, body, 0)
        plsc.subcore_barrier()
        pltpu.sync_copy(acc_sh.at[pl.ds(sid * zrows, zrows)], tbuf)
        pltpu.sync_copy(tbuf, out_hbm.at[cid].at[pl.ds(sid * zrows, zrows)])

    return scatter


def _edge_core_body(ea_ref, gs_ref, gd_ref, c_ref,
                    webd_ref, be_ref, ge_ref, bte_ref, mavg_ref,
                    w1bd_ref, gc_ref, btc_ref, woebd_ref, boe_ref,
                    e2_ref, ze_ref, m_ref, s_ref):
    i = pl.program_id(0)
    e_enc = _ln_relu_packed(ea_ref[...] @ webd_ref[...] + be_ref[...],
                            mavg_ref, ge_ref[...], bte_ref[...])
    h = e_enc @ w1bd_ref[...] + gs_ref[...] + gd_ref[...] + c_ref[...]
    e2 = _ln_relu_packed(h, mavg_ref, gc_ref[...], btc_ref[...])
    e2_ref[...] = e2
    ze = e2 @ woebd_ref[...] + boe_ref[...]
    ze_ref[...] = ze

    @pl.when(i == 0)
    def _():
        m_ref[...] = jnp.full((1, H), -jnp.inf, jnp.float32)
        s_ref[...] = jnp.zeros((1, H), jnp.float32)

    bm = jnp.max(ze, axis=0, keepdims=True)
    m_old = m_ref[...]
    m_new = jnp.maximum(m_old, bm)
    s_ref[...] = (s_ref[...] * jnp.exp(m_old - m_new)
                  + jnp.sum(jnp.exp(ze - m_new), axis=0, keepdims=True))
    m_ref[...] = m_new


def _edge_core(ea, gs, gd, c, webd, be, ge, bte, mavg, w1bd, gc, btc,
               woebd, boe, interpret=False):
    rows = ea.shape[0]
    blk = _EDGE_PBLOCK
    grid = rows // blk
    row_spec = pl.BlockSpec((blk, _LANES), lambda i: (i, 0))
    full = lambda a: pl.BlockSpec(a.shape, lambda i: tuple(0 for _ in a.shape))
    out_shape = [
        jax.ShapeDtypeStruct((rows, _LANES), jnp.float32),
        jax.ShapeDtypeStruct((rows, H), jnp.float32),
        jax.ShapeDtypeStruct((1, H), jnp.float32),
        jax.ShapeDtypeStruct((1, H), jnp.float32),
    ]
    return pl.pallas_call(
        _edge_core_body,
        grid=(grid,),
        in_specs=[row_spec, row_spec, row_spec, full(c), full(webd), full(be),
                  full(ge), full(bte), full(mavg), full(w1bd), full(gc),
                  full(btc), full(woebd), full(boe)],
        out_specs=[row_spec,
                   pl.BlockSpec((blk, H), lambda i: (i, 0)),
                   pl.BlockSpec((1, H), lambda i: (0, 0)),
                   pl.BlockSpec((1, H), lambda i: (0, 0))],
        out_shape=out_shape,
        interpret=interpret,
    )(ea, gs, gd, c, webd, be, ge, bte, mavg, w1bd, gc, btc, woebd, boe)


def _node_core_body(n_ref, a0_ref, a1_ref, wnnbd_ref, wnabd_ref, bcn_ref,
                    gcn_ref, btcn_ref, mavg_ref, wonbd_ref, bon_ref,
                    zn_ref, m_ref, s_ref):
    agg = a0_ref[...] + a1_ref[...]
    h = n_ref[...] @ wnnbd_ref[...] + agg @ wnabd_ref[...] + bcn_ref[...]
    n2 = _ln_relu_packed(h, mavg_ref, gcn_ref[...], btcn_ref[...])
    zn = n2 @ wonbd_ref[...] + bon_ref[...]
    zn_ref[...] = zn
    m = jnp.max(zn, axis=0, keepdims=True)
    m_ref[...] = m
    s_ref[...] = jnp.sum(jnp.exp(zn - m), axis=0, keepdims=True)


def _node_core(n, a0, a1, wnnbd, wnabd, bcn, gcn, btcn, mavg, wonbd, bon,
               interpret=False):
    rows = n.shape[0]
    out_shape = [
        jax.ShapeDtypeStruct((rows, H), jnp.float32),
        jax.ShapeDtypeStruct((1, H), jnp.float32),
        jax.ShapeDtypeStruct((1, H), jnp.float32),
    ]
    return pl.pallas_call(_node_core_body, out_shape=out_shape,
                          interpret=interpret)(
        n, a0, a1, wnnbd, wnabd, bcn, gcn, btcn, mavg, wonbd, bon)


def _finish_softmax(z_packed, mg, sg, n_rows):
    m8 = mg.reshape(_PACK, 2)
    s8 = sg.reshape(_PACK, 2)
    m2 = jnp.max(m8, axis=0)
    s2 = jnp.sum(s8 * jnp.exp(m8 - m2[None, :]), axis=0)
    mt = jnp.tile(m2, _PACK)[None, :]
    rt = jnp.tile(1.0 / s2, _PACK)[None, :]
    out_p = jnp.exp(z_packed - mt) * rt
    return jnp.reshape(out_p, (n_rows, 2))


def kernel(x, edge_index, edge_attr, u, params):
    n_nodes, n_edges = x.shape[0], edge_attr.shape[0]
    src = edge_index[0].astype(jnp.int32)
    dst = edge_index[1].astype(jnp.int32)
    f32 = jnp.float32
    r = lambda v: jnp.reshape(v, (1, -1))
    eye8 = jnp.eye(_PACK, dtype=f32)
    bd = lambda w: jnp.kron(eye8, w)
    t8 = lambda v: jnp.tile(jnp.reshape(v, (1, -1)), (1, _PACK))
    mavg = jnp.kron(eye8, jnp.full((H, H), 1.0 / H, f32))

    pe, pn, pg = params["enc_e"], params["enc_n"], params["enc_g"]
    ce, cn = params["core_e"], params["core_n"]
    oe, on = params["out_e"], params["out_n"]
    w1, w2, w3, w4 = (ce["W"][0:16], ce["W"][16:32], ce["W"][32:48],
                      ce["W"][48:64])
    wnn, wna = cn["W"][0:16], cn["W"][16:32]

    xr = jnp.reshape(x, (n_nodes // _PACK, _PACK * x.shape[1]))
    n_p, p2_p, p3_p, c_t = _node_encode(
        xr, u, bd(pn["W"]), t8(pn["b"]), t8(pn["g"]), t8(pn["bt"]), mavg,
        bd(w2), bd(w3),
        pg["W"], r(pg["b"]), r(pg["g"]), r(pg["bt"]),
        jnp.tile(w4, (1, _PACK)), t8(ce["b"]))

    p2 = jnp.reshape(p2_p, (n_nodes, H))
    p3 = jnp.reshape(p3_p, (n_nodes, H))
    ea_1d = jnp.reshape(edge_attr, (-1,))
    gs, gd, ea_lin = _make_gather(n_edges)(p2, p3, src, dst, ea_1d)

    ea_p = jnp.reshape(ea_lin, (n_edges // _PACK, _LANES))
    gs_p = jnp.reshape(gs, (n_edges // _PACK, _LANES))
    gd_p = jnp.reshape(gd, (n_edges // _PACK, _LANES))
    e2_p, ze_p, mg_e, sg_e = _edge_core(
        ea_p, gs_p, gd_p, c_t, bd(pe["W"]), t8(pe["b"]), t8(pe["g"]),
        t8(pe["bt"]), mavg, bd(w1), t8(ce["g"]), t8(ce["bt"]),
        bd(oe["W"]), t8(oe["b"]))

    edge_out = _finish_softmax(ze_p, mg_e, sg_e, n_edges)

    e2 = jnp.reshape(e2_p, (n_edges, H))
    aggp = _make_scatter(n_edges, n_nodes)(e2, dst)
    agg_p = jnp.reshape(aggp, (2, n_nodes // _PACK, _LANES))

    zn_p, mg_n, sg_n = _node_core(
        n_p, agg_p[0], agg_p[1], bd(wnn), bd(wna), t8(cn["b"]), t8(cn["g"]),
        t8(cn["bt"]), mavg, bd(on["W"]), t8(on["b"]))

    node_out = _finish_softmax(zn_p, mg_n, sg_n, n_nodes)

    glob_out = jnp.ones((1, 1), f32)
    return edge_out, node_out, glob_out


# speedup vs baseline: 6.5913x; 1.3897x over previous
import functools

import jax
import jax.numpy as jnp
from jax import lax
from jax.experimental import pallas as pl
from jax.experimental.pallas import tpu as pltpu
from jax.experimental.pallas import tpu_sc as plsc

H = 16
_PACK = 8
_LANES = _PACK * H
_EDGE_PBLOCK = 5000


def _ln_relu_packed(h, m_ref, g_t, bt_t):
    mavg = m_ref[...]
    mu = h @ mavg
    d = h - mu
    var = (d * d) @ mavg
    return jax.nn.relu(d * lax.rsqrt(var + 1e-5) * g_t + bt_t)


def _node_encode_body(xr_ref, u_ref, wnbd_ref, bn_ref, gn_ref, btn_ref,
                      mavg_ref, w2bd_ref, w3bd_ref,
                      wg_ref, bg_ref, gg_ref, btg_ref, w4t_ref, bce_ref,
                      n_ref, p2_ref, p3_ref, c_ref):
    h = xr_ref[...] @ wnbd_ref[...] + bn_ref[...]
    n = _ln_relu_packed(h, mavg_ref, gn_ref[...], btn_ref[...])
    n_ref[...] = n
    p2_ref[...] = n @ w2bd_ref[...]
    p3_ref[...] = n @ w3bd_ref[...]
    hg = u_ref[...] @ wg_ref[...] + bg_ref[...]
    mu = jnp.mean(hg, axis=-1, keepdims=True)
    var = jnp.mean((hg - mu) ** 2, axis=-1, keepdims=True)
    g = jax.nn.relu((hg - mu) * lax.rsqrt(var + 1e-5) * gg_ref[...]
                    + btg_ref[...])
    c_ref[...] = g @ w4t_ref[...] + bce_ref[...]


def _node_encode(xr, u, wnbd, bn, gn, btn, mavg, w2bd, w3bd,
                 wg, bg, gg, btg, w4t, bce, interpret=False):
    rows = xr.shape[0]
    out_shape = [
        jax.ShapeDtypeStruct((rows, _LANES), jnp.float32),
        jax.ShapeDtypeStruct((rows, _LANES), jnp.float32),
        jax.ShapeDtypeStruct((rows, _LANES), jnp.float32),
        jax.ShapeDtypeStruct((1, _LANES), jnp.float32),
    ]
    return pl.pallas_call(_node_encode_body, out_shape=out_shape,
                          interpret=interpret)(
        xr, u, wnbd, bn, gn, btn, mavg, w2bd, w3bd,
        wg, bg, gg, btg, w4t, bce)


@functools.cache
def _make_gather(n_edges):
    nw = 32
    per = n_edges // nw
    ch = 2000
    nch = per // ch
    mesh = plsc.VectorSubcoreMesh(core_axis_name="c", subcore_axis_name="s")

    @functools.partial(
        pl.kernel, mesh=mesh,
        compiler_params=pltpu.CompilerParams(use_tc_tiling_on_sc=False),
        out_type=[jax.ShapeDtypeStruct((n_edges, H), jnp.float32),
                  jax.ShapeDtypeStruct((n_edges, H), jnp.float32),
                  jax.ShapeDtypeStruct((n_edges // _PACK, _LANES), jnp.float32)],
        scratch_types=[pltpu.VMEM((ch,), jnp.int32),
                       pltpu.VMEM((ch,), jnp.int32),
                       pltpu.VMEM((ch, H), jnp.float32),
                       pltpu.VMEM((ch, H), jnp.float32),
                       pltpu.VMEM((ch // _PACK, _LANES), jnp.float32),
                       pltpu.SemaphoreType.DMA,
                       pltpu.SemaphoreType.DMA,
                       pltpu.SemaphoreType.DMA],
    )
    def gather(p2_hbm, p3_hbm, src_hbm, dst_hbm, ea_hbm,
               gs_hbm, gd_hbm, ea_out_hbm,
               idxa, idxb, rowsa, rowsb, rowse, sema, semb, seme):
        wid = lax.axis_index("s") * 2 + lax.axis_index("c")
        base = wid * per

        def body(c, carry):
            off = base + c * ch
            pltpu.sync_copy(src_hbm.at[pl.ds(off, ch)], idxa)
            pltpu.sync_copy(dst_hbm.at[pl.ds(off, ch)], idxb)
            cpa = pltpu.async_copy(p2_hbm.at[idxa], rowsa, sema)
            cpb = pltpu.async_copy(p3_hbm.at[idxb], rowsb, semb)
            cpe = pltpu.async_copy(ea_hbm.at[pl.ds(off // _PACK, ch // _PACK)],
                                   rowse, seme)
            cpa.wait()
            pltpu.sync_copy(rowsa, gs_hbm.at[pl.ds(off, ch)])
            cpb.wait()
            pltpu.sync_copy(rowsb, gd_hbm.at[pl.ds(off, ch)])
            cpe.wait()
            pltpu.sync_copy(rowse, ea_out_hbm.at[pl.ds(off // _PACK, ch // _PACK)])
            return carry

        lax.fori_loop(0, nch, body, 0)

    return gather


@functools.cache
def _make_scatter(n_edges, n_nodes):
    nw = 32
    per = n_edges // nw
    ch = 2000
    nch = per // ch
    zrows = n_nodes // 16
    mesh = plsc.VectorSubcoreMesh(core_axis_name="c", subcore_axis_name="s")

    @functools.partial(
        pl.kernel, mesh=mesh,
        compiler_params=pltpu.CompilerParams(use_tc_tiling_on_sc=False),
        out_type=jax.ShapeDtypeStruct((2, n_nodes, H), jnp.float32),
        scratch_types=[pltpu.VMEM((ch,), jnp.int32),
                       pltpu.VMEM((ch, H), jnp.float32),
                       pltpu.VMEM((zrows, H), jnp.float32),
                       pltpu.VMEM_SHARED((n_nodes, H), jnp.float32),
                       pltpu.SemaphoreType.DMA],
    )
    def scatter(e2_hbm, dst_hbm, out_hbm, idx, rows, tbuf, acc_sh, sem):
        cid = lax.axis_index("c")
        sid = lax.axis_index("s")

        def zero_row(r, carry):
            tbuf[r, :] = jnp.zeros((H,), jnp.float32)
            return carry

        lax.fori_loop(0, zrows, zero_row, 0)
        pltpu.sync_copy(tbuf, acc_sh.at[pl.ds(sid * zrows, zrows)])
        plsc.subcore_barrier()

        wid = sid * 2 + cid
        base = wid * per

        def body(c, carry):
            off = base + c * ch
            pltpu.sync_copy(dst_hbm.at[pl.ds(off, ch)], idx)
            cp = pltpu.async_copy(e2_hbm.at[pl.ds(off, ch)], rows, sem)
            cp.wait()
            pltpu.sync_copy(rows, acc_sh.at[idx], add=True)
            return carry

        lax.fori_loop(0, nch, body, 0)
        plsc.subcore_barrier()
        pltpu.sync_copy(acc_sh.at[pl.ds(sid * zrows, zrows)], tbuf)
        pltpu.sync_copy(tbuf, out_hbm.at[cid].at[pl.ds(sid * zrows, zrows)])

    return scatter


def _edge_core_body(ea_ref, gs_ref, gd_ref, c_ref,
                    webd_ref, be_ref, ge_ref, bte_ref, mavg_ref,
                    w1bd_ref, gc_ref, btc_ref, woebd_ref, boe_ref,
                    e2_ref, ze_ref, m_ref, s_ref):
    i = pl.program_id(0)
    e_enc = _ln_relu_packed(ea_ref[...] @ webd_ref[...] + be_ref[...],
                            mavg_ref, ge_ref[...], bte_ref[...])
    h = e_enc @ w1bd_ref[...] + gs_ref[...] + gd_ref[...] + c_ref[...]
    e2 = _ln_relu_packed(h, mavg_ref, gc_ref[...], btc_ref[...])
    e2_ref[...] = e2
    ze = e2 @ woebd_ref[...] + boe_ref[...]
    ze_ref[...] = jnp.transpose(ze)[None]

    @pl.when(i == 0)
    def _():
        m_ref[...] = jnp.full((1, H), -jnp.inf, jnp.float32)
        s_ref[...] = jnp.zeros((1, H), jnp.float32)

    bm = jnp.max(ze, axis=0, keepdims=True)
    m_old = m_ref[...]
    m_new = jnp.maximum(m_old, bm)
    s_ref[...] = (s_ref[...] * jnp.exp(m_old - m_new)
                  + jnp.sum(jnp.exp(ze - m_new), axis=0, keepdims=True))
    m_ref[...] = m_new


def _edge_core(ea, gs, gd, c, webd, be, ge, bte, mavg, w1bd, gc, btc,
               woebd, boe, interpret=False):
    rows = ea.shape[0]
    blk = _EDGE_PBLOCK
    grid = rows // blk
    row_spec = pl.BlockSpec((blk, _LANES), lambda i: (i, 0))
    full = lambda a: pl.BlockSpec(a.shape, lambda i: tuple(0 for _ in a.shape))
    out_shape = [
        jax.ShapeDtypeStruct((rows, _LANES), jnp.float32),
        jax.ShapeDtypeStruct((grid, H, blk), jnp.float32),
        jax.ShapeDtypeStruct((1, H), jnp.float32),
        jax.ShapeDtypeStruct((1, H), jnp.float32),
    ]
    return pl.pallas_call(
        _edge_core_body,
        grid=(grid,),
        in_specs=[row_spec, row_spec, row_spec, full(c), full(webd), full(be),
                  full(ge), full(bte), full(mavg), full(w1bd), full(gc),
                  full(btc), full(woebd), full(boe)],
        out_specs=[row_spec, pl.BlockSpec((1, H, blk), lambda i: (i, 0, 0)),
                   pl.BlockSpec((1, H), lambda i: (0, 0)),
                   pl.BlockSpec((1, H), lambda i: (0, 0))],
        out_shape=out_shape,
        interpret=interpret,
    )(ea, gs, gd, c, webd, be, ge, bte, mavg, w1bd, gc, btc, woebd, boe)


def _node_core_body(n_ref, a0_ref, a1_ref, wnnbd_ref, wnabd_ref, bcn_ref,
                    gcn_ref, btcn_ref, mavg_ref, wonbd_ref, bon_ref,
                    zn_ref, m_ref, s_ref):
    agg = a0_ref[...] + a1_ref[...]
    h = n_ref[...] @ wnnbd_ref[...] + agg @ wnabd_ref[...] + bcn_ref[...]
    n2 = _ln_relu_packed(h, mavg_ref, gcn_ref[...], btcn_ref[...])
    zn = n2 @ wonbd_ref[...] + bon_ref[...]
    zn_ref[...] = jnp.transpose(zn)
    m = jnp.max(zn, axis=0, keepdims=True)
    m_ref[...] = m
    s_ref[...] = jnp.sum(jnp.exp(zn - m), axis=0, keepdims=True)


def _node_core(n, a0, a1, wnnbd, wnabd, bcn, gcn, btcn, mavg, wonbd, bon,
               interpret=False):
    rows = n.shape[0]
    out_shape = [
        jax.ShapeDtypeStruct((H, rows), jnp.float32),
        jax.ShapeDtypeStruct((1, H), jnp.float32),
        jax.ShapeDtypeStruct((1, H), jnp.float32),
    ]
    return pl.pallas_call(_node_core_body, out_shape=out_shape,
                          interpret=interpret)(
        n, a0, a1, wnnbd, wnabd, bcn, gcn, btcn, mavg, wonbd, bon)


def _finish_softmax(z_t, mg, sg, n_rows):
    m8 = mg[0].reshape(_PACK, 2)
    s8 = sg[0].reshape(_PACK, 2)
    m2 = jnp.max(m8, axis=0)
    s2 = jnp.sum(s8 * jnp.exp(m8 - m2[None, :]), axis=0)
    g, _, b = z_t.shape
    z5 = z_t.reshape(g, _PACK, 2, b)
    ex = (jnp.exp(z5 - m2[None, None, :, None])
          * (1.0 / s2)[None, None, :, None])
    return jnp.transpose(ex, (0, 3, 1, 2)).reshape(n_rows, 2)


def kernel(x, edge_index, edge_attr, u, params):
    n_nodes, n_edges = x.shape[0], edge_attr.shape[0]
    src = edge_index[0].astype(jnp.int32)
    dst = edge_index[1].astype(jnp.int32)
    f32 = jnp.float32
    r = lambda v: jnp.reshape(v, (1, -1))
    eye8 = jnp.eye(_PACK, dtype=f32)
    bd = lambda w: jnp.kron(eye8, w)
    t8 = lambda v: jnp.tile(jnp.reshape(v, (1, -1)), (1, _PACK))
    mavg = jnp.kron(eye8, jnp.full((H, H), 1.0 / H, f32))

    pe, pn, pg = params["enc_e"], params["enc_n"], params["enc_g"]
    ce, cn = params["core_e"], params["core_n"]
    oe, on = params["out_e"], params["out_n"]
    w1, w2, w3, w4 = (ce["W"][0:16], ce["W"][16:32], ce["W"][32:48],
                      ce["W"][48:64])
    wnn, wna = cn["W"][0:16], cn["W"][16:32]

    xr = jnp.reshape(x, (n_nodes // _PACK, _PACK * x.shape[1]))
    n_p, p2_p, p3_p, c_t = _node_encode(
        xr, u, bd(pn["W"]), t8(pn["b"]), t8(pn["g"]), t8(pn["bt"]), mavg,
        bd(w2), bd(w3),
        pg["W"], r(pg["b"]), r(pg["g"]), r(pg["bt"]),
        jnp.tile(w4, (1, _PACK)), t8(ce["b"]))

    p2 = jnp.reshape(p2_p, (n_nodes, H))
    p3 = jnp.reshape(p3_p, (n_nodes, H))
    ea_grp = jnp.reshape(edge_attr, (n_edges // _PACK, _LANES))
    gs, gd, ea_p = _make_gather(n_edges)(p2, p3, src, dst, ea_grp)

    gs_p = jnp.reshape(gs, (n_edges // _PACK, _LANES))
    gd_p = jnp.reshape(gd, (n_edges // _PACK, _LANES))
    e2_p, ze_t, mg_e, sg_e = _edge_core(
        ea_p, gs_p, gd_p, c_t, bd(pe["W"]), t8(pe["b"]), t8(pe["g"]),
        t8(pe["bt"]), mavg, bd(w1), t8(ce["g"]), t8(ce["bt"]),
        bd(oe["W"]), t8(oe["b"]))

    edge_out = _finish_softmax(ze_t, mg_e, sg_e, n_edges)

    e2 = jnp.reshape(e2_p, (n_edges, H))
    aggp = _make_scatter(n_edges, n_nodes)(e2, dst)
    agg_p = jnp.reshape(aggp, (2, n_nodes // _PACK, _LANES))

    zn_t, mg_n, sg_n = _node_core(
        n_p, agg_p[0], agg_p[1], bd(wnn), bd(wna), t8(cn["b"]), t8(cn["g"]),
        t8(cn["bt"]), mavg, bd(on["W"]), t8(on["b"]))

    node_out = _finish_softmax(zn_t[None], mg_n, sg_n, n_nodes)

    glob_out = jnp.ones((1, 1), f32)
    return edge_out, node_out, glob_out


# speedup vs baseline: 6.8308x; 1.0363x over previous
import functools

import jax
import jax.numpy as jnp
from jax import lax
from jax.experimental import pallas as pl
from jax.experimental.pallas import tpu as pltpu
from jax.experimental.pallas import tpu_sc as plsc

H = 16
_PACK = 8
_LANES = _PACK * H
_EDGE_PBLOCK = 5000


def _ln_relu_packed(h, m_ref, g_t, bt_t):
    mavg = m_ref[...]
    mu = h @ mavg
    d = h - mu
    var = (d * d) @ mavg
    return jax.nn.relu(d * lax.rsqrt(var + 1e-5) * g_t + bt_t)


def _node_encode_body(xr_ref, u_ref, wnbd_ref, bn_ref, gn_ref, btn_ref,
                      mavg_ref, w2bd_ref, w3bd_ref,
                      wg_ref, bg_ref, gg_ref, btg_ref, w4t_ref, bce_ref,
                      n_ref, p2_ref, p3_ref, c_ref):
    h = xr_ref[...] @ wnbd_ref[...] + bn_ref[...]
    n = _ln_relu_packed(h, mavg_ref, gn_ref[...], btn_ref[...])
    n_ref[...] = n
    p2_ref[...] = n @ w2bd_ref[...]
    p3_ref[...] = n @ w3bd_ref[...]
    hg = u_ref[...] @ wg_ref[...] + bg_ref[...]
    mu = jnp.mean(hg, axis=-1, keepdims=True)
    var = jnp.mean((hg - mu) ** 2, axis=-1, keepdims=True)
    g = jax.nn.relu((hg - mu) * lax.rsqrt(var + 1e-5) * gg_ref[...]
                    + btg_ref[...])
    c_ref[...] = g @ w4t_ref[...] + bce_ref[...]


def _node_encode(xr, u, wnbd, bn, gn, btn, mavg, w2bd, w3bd,
                 wg, bg, gg, btg, w4t, bce, interpret=False):
    rows = xr.shape[0]
    out_shape = [
        jax.ShapeDtypeStruct((rows, _LANES), jnp.float32),
        jax.ShapeDtypeStruct((rows, _LANES), jnp.float32),
        jax.ShapeDtypeStruct((rows, _LANES), jnp.float32),
        jax.ShapeDtypeStruct((1, _LANES), jnp.float32),
    ]
    return pl.pallas_call(_node_encode_body, out_shape=out_shape,
                          interpret=interpret)(
        xr, u, wnbd, bn, gn, btn, mavg, w2bd, w3bd,
        wg, bg, gg, btg, w4t, bce)


@functools.cache
def _make_gather(n_edges):
    nw = 32
    per = n_edges // nw
    ch = 2000
    nch = per // ch
    mesh = plsc.VectorSubcoreMesh(core_axis_name="c", subcore_axis_name="s")

    @functools.partial(
        pl.kernel, mesh=mesh,
        compiler_params=pltpu.CompilerParams(use_tc_tiling_on_sc=False,
                                             needs_layout_passes=False),
        out_type=[jax.ShapeDtypeStruct((n_edges, H), jnp.float32),
                  jax.ShapeDtypeStruct((n_edges, H), jnp.float32),
                  jax.ShapeDtypeStruct((n_edges, H), jnp.float32)],
        scratch_types=[pltpu.VMEM((ch,), jnp.int32),
                       pltpu.VMEM((ch,), jnp.int32),
                       pltpu.VMEM((ch, H), jnp.float32),
                       pltpu.VMEM((ch, H), jnp.float32),
                       pltpu.VMEM((H, ch), jnp.float32),
                       pltpu.SemaphoreType.DMA,
                       pltpu.SemaphoreType.DMA,
                       pltpu.SemaphoreType.DMA],
    )
    def gather(p2_hbm, p3_hbm, src_hbm, dst_hbm, eat_hbm,
               gs_hbm, gd_hbm, ea_out_hbm,
               idxa, idxb, rowsa, rowsb, sbuf, sema, semb, seme):
        wid = lax.axis_index("s") * 2 + lax.axis_index("c")
        base = wid * per
        lanes = lax.iota(jnp.int32, 16)

        def body(c, carry):
            off = base + c * ch
            pltpu.sync_copy(src_hbm.at[pl.ds(off, ch)], idxa)
            pltpu.sync_copy(dst_hbm.at[pl.ds(off, ch)], idxb)
            cpa = pltpu.async_copy(p2_hbm.at[idxa], rowsa, sema)
            cpb = pltpu.async_copy(p3_hbm.at[idxb], rowsb, semb)
            cpe = pltpu.async_copy(eat_hbm.at[:, pl.ds(off, ch)], sbuf, seme)
            cpa.wait()
            pltpu.sync_copy(rowsa, gs_hbm.at[pl.ds(off, ch)])
            cpb.wait()
            pltpu.sync_copy(rowsb, gd_hbm.at[pl.ds(off, ch)])
            cpe.wait()

            def trans8(e8, carry2):
                e = e8 * 8
                for t in range(8):
                    col = jnp.full((16,), e + t, jnp.int32)
                    rowsa[e + t, :] = plsc.load_gather(sbuf, [lanes, col])
                return carry2

            lax.fori_loop(0, ch // 8, trans8, 0)
            pltpu.sync_copy(rowsa, ea_out_hbm.at[pl.ds(off, ch)])
            return carry

        lax.fori_loop(0, nch, body, 0)

    return gather


@functools.cache
def _make_scatter(n_edges, n_nodes):
    nw = 32
    per = n_edges // nw
    ch = 2000
    nch = per // ch
    zrows = n_nodes // 16
    mesh = plsc.VectorSubcoreMesh(core_axis_name="c", subcore_axis_name="s")

    @functools.partial(
        pl.kernel, mesh=mesh,
        compiler_params=pltpu.CompilerParams(use_tc_tiling_on_sc=False),
        out_type=jax.ShapeDtypeStruct((2, n_nodes, H), jnp.float32),
        scratch_types=[pltpu.VMEM((ch,), jnp.int32),
                       pltpu.VMEM((ch, H), jnp.float32),
                       pltpu.VMEM((zrows, H), jnp.float32),
                       pltpu.VMEM_SHARED((n_nodes, H), jnp.float32),
                       pltpu.SemaphoreType.DMA],
    )
    def scatter(e2_hbm, dst_hbm, out_hbm, idx, rows, tbuf, acc_sh, sem):
        cid = lax.axis_index("c")
        sid = lax.axis_index("s")

        def zero_row(r, carry):
            tbuf[r, :] = jnp.zeros((H,), jnp.float32)
            return carry

        lax.fori_loop(0, zrows, zero_row, 0)
        pltpu.sync_copy(tbuf, acc_sh.at[pl.ds(sid * zrows, zrows)])
        plsc.subcore_barrier()

        wid = sid * 2 + cid
        base = wid * per

        def body(c, carry):
            off = base + c * ch
            pltpu.sync_copy(dst_hbm.at[pl.ds(off, ch)], idx)
            cp = pltpu.async_copy(e2_hbm.at[pl.ds(off, ch)], rows, sem)
            cp.wait()
            pltpu.sync_copy(rows, acc_sh.at[idx], add=True)
            return carry

        lax.fori_loop(0, nch, body, 0)
        plsc.subcore_barrier()
        pltpu.sync_copy(acc_sh.at[pl.ds(sid * zrows, zrows)], tbuf)
        pltpu.sync_copy(tbuf, out_hbm.at[cid].at[pl.ds(sid * zrows, zrows)])

    return scatter


def _edge_core_body(ea_ref, gs_ref, gd_ref, c_ref,
                    webd_ref, be_ref, ge_ref, bte_ref, mavg_ref,
                    w1bd_ref, gc_ref, btc_ref, woebd_ref, boe_ref,
                    e2_ref, ze_ref, m_ref, s_ref):
    i = pl.program_id(0)
    e_enc = _ln_relu_packed(ea_ref[...] @ webd_ref[...] + be_ref[...],
                            mavg_ref, ge_ref[...], bte_ref[...])
    h = e_enc @ w1bd_ref[...] + gs_ref[...] + gd_ref[...] + c_ref[...]
    e2 = _ln_relu_packed(h, mavg_ref, gc_ref[...], btc_ref[...])
    e2_ref[...] = e2
    ze = e2 @ woebd_ref[...] + boe_ref[...]
    ze_ref[...] = jnp.transpose(ze)[None]

    @pl.when(i == 0)
    def _():
        m_ref[...] = jnp.full((1, H), -jnp.inf, jnp.float32)
        s_ref[...] = jnp.zeros((1, H), jnp.float32)

    bm = jnp.max(ze, axis=0, keepdims=True)
    m_old = m_ref[...]
    m_new = jnp.maximum(m_old, bm)
    s_ref[...] = (s_ref[...] * jnp.exp(m_old - m_new)
                  + jnp.sum(jnp.exp(ze - m_new), axis=0, keepdims=True))
    m_ref[...] = m_new


def _edge_core(ea, gs, gd, c, webd, be, ge, bte, mavg, w1bd, gc, btc,
               woebd, boe, interpret=False):
    rows = ea.shape[0]
    blk = _EDGE_PBLOCK
    grid = rows // blk
    row_spec = pl.BlockSpec((blk, _LANES), lambda i: (i, 0))
    full = lambda a: pl.BlockSpec(a.shape, lambda i: tuple(0 for _ in a.shape))
    out_shape = [
        jax.ShapeDtypeStruct((rows, _LANES), jnp.float32),
        jax.ShapeDtypeStruct((grid, H, blk), jnp.float32),
        jax.ShapeDtypeStruct((1, H), jnp.float32),
        jax.ShapeDtypeStruct((1, H), jnp.float32),
    ]
    return pl.pallas_call(
        _edge_core_body,
        grid=(grid,),
        in_specs=[row_spec, row_spec, row_spec, full(c), full(webd), full(be),
                  full(ge), full(bte), full(mavg), full(w1bd), full(gc),
                  full(btc), full(woebd), full(boe)],
        out_specs=[row_spec, pl.BlockSpec((1, H, blk), lambda i: (i, 0, 0)),
                   pl.BlockSpec((1, H), lambda i: (0, 0)),
                   pl.BlockSpec((1, H), lambda i: (0, 0))],
        out_shape=out_shape,
        interpret=interpret,
    )(ea, gs, gd, c, webd, be, ge, bte, mavg, w1bd, gc, btc, woebd, boe)


def _node_core_body(n_ref, a0_ref, a1_ref, wnnbd_ref, wnabd_ref, bcn_ref,
                    gcn_ref, btcn_ref, mavg_ref, wonbd_ref, bon_ref,
                    zn_ref, m_ref, s_ref):
    agg = a0_ref[...] + a1_ref[...]
    h = n_ref[...] @ wnnbd_ref[...] + agg @ wnabd_ref[...] + bcn_ref[...]
    n2 = _ln_relu_packed(h, mavg_ref, gcn_ref[...], btcn_ref[...])
    zn = n2 @ wonbd_ref[...] + bon_ref[...]
    zn_ref[...] = jnp.transpose(zn)
    m = jnp.max(zn, axis=0, keepdims=True)
    m_ref[...] = m
    s_ref[...] = jnp.sum(jnp.exp(zn - m), axis=0, keepdims=True)


def _node_core(n, a0, a1, wnnbd, wnabd, bcn, gcn, btcn, mavg, wonbd, bon,
               interpret=False):
    rows = n.shape[0]
    out_shape = [
        jax.ShapeDtypeStruct((H, rows), jnp.float32),
        jax.ShapeDtypeStruct((1, H), jnp.float32),
        jax.ShapeDtypeStruct((1, H), jnp.float32),
    ]
    return pl.pallas_call(_node_core_body, out_shape=out_shape,
                          interpret=interpret)(
        n, a0, a1, wnnbd, wnabd, bcn, gcn, btcn, mavg, wonbd, bon)


def _finish_softmax(z_t, mg, sg, n_rows):
    m8 = mg[0].reshape(_PACK, 2)
    s8 = sg[0].reshape(_PACK, 2)
    m2 = jnp.max(m8, axis=0)
    s2 = jnp.sum(s8 * jnp.exp(m8 - m2[None, :]), axis=0)
    g, _, b = z_t.shape
    z5 = z_t.reshape(g, _PACK, 2, b)
    ex = (jnp.exp(z5 - m2[None, None, :, None])
          * (1.0 / s2)[None, None, :, None])
    return jnp.transpose(ex, (0, 3, 1, 2)).reshape(n_rows, 2)


def kernel(x, edge_index, edge_attr, u, params):
    n_nodes, n_edges = x.shape[0], edge_attr.shape[0]
    src = edge_index[0].astype(jnp.int32)
    dst = edge_index[1].astype(jnp.int32)
    f32 = jnp.float32
    r = lambda v: jnp.reshape(v, (1, -1))
    eye8 = jnp.eye(_PACK, dtype=f32)
    bd = lambda w: jnp.kron(eye8, w)
    t8 = lambda v: jnp.tile(jnp.reshape(v, (1, -1)), (1, _PACK))
    mavg = jnp.kron(eye8, jnp.full((H, H), 1.0 / H, f32))

    pe, pn, pg = params["enc_e"], params["enc_n"], params["enc_g"]
    ce, cn = params["core_e"], params["core_n"]
    oe, on = params["out_e"], params["out_n"]
    w1, w2, w3, w4 = (ce["W"][0:16], ce["W"][16:32], ce["W"][32:48],
                      ce["W"][48:64])
    wnn, wna = cn["W"][0:16], cn["W"][16:32]

    xr = jnp.reshape(x, (n_nodes // _PACK, _PACK * x.shape[1]))
    n_p, p2_p, p3_p, c_t = _node_encode(
        xr, u, bd(pn["W"]), t8(pn["b"]), t8(pn["g"]), t8(pn["bt"]), mavg,
        bd(w2), bd(w3),
        pg["W"], r(pg["b"]), r(pg["g"]), r(pg["bt"]),
        jnp.tile(w4, (1, _PACK)), t8(ce["b"]))

    p2 = jnp.reshape(p2_p, (n_nodes, H))
    p3 = jnp.reshape(p3_p, (n_nodes, H))
    ea_t = jnp.swapaxes(edge_attr, 0, 1)
    gs, gd, ea_lin = _make_gather(n_edges)(p2, p3, src, dst, ea_t)

    ea_p = jnp.reshape(ea_lin, (n_edges // _PACK, _LANES))
    gs_p = jnp.reshape(gs, (n_edges // _PACK, _LANES))
    gd_p = jnp.reshape(gd, (n_edges // _PACK, _LANES))
    e2_p, ze_t, mg_e, sg_e = _edge_core(
        ea_p, gs_p, gd_p, c_t, bd(pe["W"]), t8(pe["b"]), t8(pe["g"]),
        t8(pe["bt"]), mavg, bd(w1), t8(ce["g"]), t8(ce["bt"]),
        bd(oe["W"]), t8(oe["b"]))

    edge_out = _finish_softmax(ze_t, mg_e, sg_e, n_edges)

    e2 = jnp.reshape(e2_p, (n_edges, H))
    aggp = _make_scatter(n_edges, n_nodes)(e2, dst)
    agg_p = jnp.reshape(aggp, (2, n_nodes // _PACK, _LANES))

    zn_t, mg_n, sg_n = _node_core(
        n_p, agg_p[0], agg_p[1], bd(wnn), bd(wna), t8(cn["b"]), t8(cn["g"]),
        t8(cn["bt"]), mavg, bd(on["W"]), t8(on["b"]))

    node_out = _finish_softmax(zn_t[None], mg_n, sg_n, n_nodes)

    glob_out = jnp.ones((1, 1), f32)
    return edge_out, node_out, glob_out


# speedup vs baseline: 7.2933x; 1.0677x over previous
import functools

import jax
import jax.numpy as jnp
from jax import lax
from jax.experimental import pallas as pl
from jax.experimental.pallas import tpu as pltpu
from jax.experimental.pallas import tpu_sc as plsc

H = 16
_PACK = 8
_LANES = _PACK * H
_EDGE_PBLOCK = 5000


def _ln_relu_packed(h, m_ref, g_t, bt_t):
    mavg = m_ref[...]
    mu = h @ mavg
    d = h - mu
    var = (d * d) @ mavg
    return jax.nn.relu(d * lax.rsqrt(var + 1e-5) * g_t + bt_t)


def _node_encode_body(xr_ref, u_ref, wnbd_ref, bn_ref, gn_ref, btn_ref,
                      mavg_ref, w2bd_ref, w3bd_ref,
                      wg_ref, bg_ref, gg_ref, btg_ref, w4t_ref, bce_ref,
                      n_ref, p2_ref, p3_ref, c_ref):
    h = xr_ref[...] @ wnbd_ref[...] + bn_ref[...]
    n = _ln_relu_packed(h, mavg_ref, gn_ref[...], btn_ref[...])
    n_ref[...] = n
    p2_ref[...] = n @ w2bd_ref[...]
    p3_ref[...] = n @ w3bd_ref[...]
    hg = u_ref[...] @ wg_ref[...] + bg_ref[...]
    mu = jnp.mean(hg, axis=-1, keepdims=True)
    var = jnp.mean((hg - mu) ** 2, axis=-1, keepdims=True)
    g = jax.nn.relu((hg - mu) * lax.rsqrt(var + 1e-5) * gg_ref[...]
                    + btg_ref[...])
    c_ref[...] = g @ w4t_ref[...] + bce_ref[...]


def _node_encode(xr, u, wnbd, bn, gn, btn, mavg, w2bd, w3bd,
                 wg, bg, gg, btg, w4t, bce, interpret=False):
    rows = xr.shape[0]
    out_shape = [
        jax.ShapeDtypeStruct((rows, _LANES), jnp.float32),
        jax.ShapeDtypeStruct((rows, _LANES), jnp.float32),
        jax.ShapeDtypeStruct((rows, _LANES), jnp.float32),
        jax.ShapeDtypeStruct((1, _LANES), jnp.float32),
    ]
    return pl.pallas_call(_node_encode_body, out_shape=out_shape,
                          interpret=interpret)(
        xr, u, wnbd, bn, gn, btn, mavg, w2bd, w3bd,
        wg, bg, gg, btg, w4t, bce)


@functools.cache
def _make_gather(n_edges):
    nw = 32
    per = n_edges // nw
    ch = 1000
    nch = per // ch
    mesh = plsc.VectorSubcoreMesh(core_axis_name="c", subcore_axis_name="s")

    @functools.partial(
        pl.kernel, mesh=mesh,
        compiler_params=pltpu.CompilerParams(use_tc_tiling_on_sc=False,
                                             needs_layout_passes=False),
        out_type=[jax.ShapeDtypeStruct((n_edges, H), jnp.float32),
                  jax.ShapeDtypeStruct((n_edges, H), jnp.float32),
                  jax.ShapeDtypeStruct((n_edges, H), jnp.float32)],
        scratch_types=[pltpu.VMEM((ch,), jnp.int32),
                       pltpu.VMEM((ch,), jnp.int32),
                       pltpu.VMEM((ch, H), jnp.float32),
                       pltpu.VMEM((ch, H), jnp.float32),
                       pltpu.VMEM((H, ch), jnp.float32),
                       pltpu.VMEM((ch, H), jnp.float32),
                       pltpu.SemaphoreType.DMA,
                       pltpu.SemaphoreType.DMA,
                       pltpu.SemaphoreType.DMA],
    )
    def gather(p2_hbm, p3_hbm, src_hbm, dst_hbm, eat_hbm,
               gs_hbm, gd_hbm, ea_out_hbm,
               idxa, idxb, rowsa, rowsb, sbuf, rowse, sema, semb, seme):
        wid = lax.axis_index("s") * 2 + lax.axis_index("c")
        base = wid * per
        lanes = lax.iota(jnp.int32, 16)

        def body(c, carry):
            off = base + c * ch
            pltpu.sync_copy(src_hbm.at[pl.ds(off, ch)], idxa)
            pltpu.sync_copy(dst_hbm.at[pl.ds(off, ch)], idxb)
            cpa = pltpu.async_copy(p2_hbm.at[idxa], rowsa, sema)
            cpb = pltpu.async_copy(p3_hbm.at[idxb], rowsb, semb)
            cpe = pltpu.async_copy(eat_hbm.at[:, pl.ds(off, ch)], sbuf, seme)
            cpe.wait()

            def trans8(e8, carry2):
                e = e8 * 8
                for t in range(8):
                    col = jnp.full((16,), e + t, jnp.int32)
                    rowse[e + t, :] = plsc.load_gather(sbuf, [lanes, col])
                return carry2

            lax.fori_loop(0, ch // 8, trans8, 0)
            pltpu.sync_copy(rowse, ea_out_hbm.at[pl.ds(off, ch)])
            cpa.wait()
            pltpu.sync_copy(rowsa, gs_hbm.at[pl.ds(off, ch)])
            cpb.wait()
            pltpu.sync_copy(rowsb, gd_hbm.at[pl.ds(off, ch)])
            return carry

        lax.fori_loop(0, nch, body, 0)

    return gather


@functools.cache
def _make_scatter(n_edges, n_nodes):
    nw = 32
    per = n_edges // nw
    ch = 2000
    nch = per // ch
    zrows = n_nodes // 16
    mesh = plsc.VectorSubcoreMesh(core_axis_name="c", subcore_axis_name="s")

    @functools.partial(
        pl.kernel, mesh=mesh,
        compiler_params=pltpu.CompilerParams(use_tc_tiling_on_sc=False),
        out_type=jax.ShapeDtypeStruct((2, n_nodes, H), jnp.float32),
        scratch_types=[pltpu.VMEM((ch,), jnp.int32),
                       pltpu.VMEM((ch, H), jnp.float32),
                       pltpu.VMEM((zrows, H), jnp.float32),
                       pltpu.VMEM_SHARED((n_nodes, H), jnp.float32),
                       pltpu.SemaphoreType.DMA],
    )
    def scatter(e2_hbm, dst_hbm, out_hbm, idx, rows, tbuf, acc_sh, sem):
        cid = lax.axis_index("c")
        sid = lax.axis_index("s")

        def zero_row(r, carry):
            tbuf[r, :] = jnp.zeros((H,), jnp.float32)
            return carry

        lax.fori_loop(0, zrows, zero_row, 0)
        pltpu.sync_copy(tbuf, acc_sh.at[pl.ds(sid * zrows, zrows)])
        plsc.subcore_barrier()

        wid = sid * 2 + cid
        base = wid * per

        def body(c, carry):
            off = base + c * ch
            pltpu.sync_copy(dst_hbm.at[pl.ds(off, ch)], idx)
            cp = pltpu.async_copy(e2_hbm.at[pl.ds(off, ch)], rows, sem)
            cp.wait()
            pltpu.sync_copy(rows, acc_sh.at[idx], add=True)
            return carry

        lax.fori_loop(0, nch, body, 0)
        plsc.subcore_barrier()
        pltpu.sync_copy(acc_sh.at[pl.ds(sid * zrows, zrows)], tbuf)
        pltpu.sync_copy(tbuf, out_hbm.at[cid].at[pl.ds(sid * zrows, zrows)])

    return scatter


def _edge_core_body(ea_ref, gs_ref, gd_ref, c_ref,
                    webd_ref, be_ref, ge_ref, bte_ref, mavg_ref,
                    w1bd_ref, gc_ref, btc_ref, woebd_ref, boe_ref,
                    e2_ref, ze_ref, m_ref, s_ref):
    i = pl.program_id(0)
    e_enc = _ln_relu_packed(ea_ref[...] @ webd_ref[...] + be_ref[...],
                            mavg_ref, ge_ref[...], bte_ref[...])
    h = e_enc @ w1bd_ref[...] + gs_ref[...] + gd_ref[...] + c_ref[...]
    e2 = _ln_relu_packed(h, mavg_ref, gc_ref[...], btc_ref[...])
    e2_ref[...] = e2
    ze = e2 @ woebd_ref[...] + boe_ref[...]
    ze_ref[...] = jnp.transpose(ze)[None]

    @pl.when(i == 0)
    def _():
        m_ref[...] = jnp.full((1, H), -jnp.inf, jnp.float32)
        s_ref[...] = jnp.zeros((1, H), jnp.float32)

    bm = jnp.max(ze, axis=0, keepdims=True)
    m_old = m_ref[...]
    m_new = jnp.maximum(m_old, bm)
    s_ref[...] = (s_ref[...] * jnp.exp(m_old - m_new)
                  + jnp.sum(jnp.exp(ze - m_new), axis=0, keepdims=True))
    m_ref[...] = m_new


def _edge_core(ea, gs, gd, c, webd, be, ge, bte, mavg, w1bd, gc, btc,
               woebd, boe, interpret=False):
    rows = ea.shape[0]
    blk = _EDGE_PBLOCK
    grid = rows // blk
    row_spec = pl.BlockSpec((blk, _LANES), lambda i: (i, 0))
    full = lambda a: pl.BlockSpec(a.shape, lambda i: tuple(0 for _ in a.shape))
    out_shape = [
        jax.ShapeDtypeStruct((rows, _LANES), jnp.float32),
        jax.ShapeDtypeStruct((grid, H, blk), jnp.float32),
        jax.ShapeDtypeStruct((1, H), jnp.float32),
        jax.ShapeDtypeStruct((1, H), jnp.float32),
    ]
    return pl.pallas_call(
        _edge_core_body,
        grid=(grid,),
        in_specs=[row_spec, row_spec, row_spec, full(c), full(webd), full(be),
                  full(ge), full(bte), full(mavg), full(w1bd), full(gc),
                  full(btc), full(woebd), full(boe)],
        out_specs=[row_spec, pl.BlockSpec((1, H, blk), lambda i: (i, 0, 0)),
                   pl.BlockSpec((1, H), lambda i: (0, 0)),
                   pl.BlockSpec((1, H), lambda i: (0, 0))],
        out_shape=out_shape,
        interpret=interpret,
    )(ea, gs, gd, c, webd, be, ge, bte, mavg, w1bd, gc, btc, woebd, boe)


def _node_core_body(n_ref, a0_ref, a1_ref, wnnbd_ref, wnabd_ref, bcn_ref,
                    gcn_ref, btcn_ref, mavg_ref, wonbd_ref, bon_ref,
                    zn_ref, m_ref, s_ref):
    agg = a0_ref[...] + a1_ref[...]
    h = n_ref[...] @ wnnbd_ref[...] + agg @ wnabd_ref[...] + bcn_ref[...]
    n2 = _ln_relu_packed(h, mavg_ref, gcn_ref[...], btcn_ref[...])
    zn = n2 @ wonbd_ref[...] + bon_ref[...]
    zn_ref[...] = jnp.transpose(zn)
    m = jnp.max(zn, axis=0, keepdims=True)
    m_ref[...] = m
    s_ref[...] = jnp.sum(jnp.exp(zn - m), axis=0, keepdims=True)


def _node_core(n, a0, a1, wnnbd, wnabd, bcn, gcn, btcn, mavg, wonbd, bon,
               interpret=False):
    rows = n.shape[0]
    out_shape = [
        jax.ShapeDtypeStruct((H, rows), jnp.float32),
        jax.ShapeDtypeStruct((1, H), jnp.float32),
        jax.ShapeDtypeStruct((1, H), jnp.float32),
    ]
    return pl.pallas_call(_node_core_body, out_shape=out_shape,
                          interpret=interpret)(
        n, a0, a1, wnnbd, wnabd, bcn, gcn, btcn, mavg, wonbd, bon)


def _finish_softmax(z_t, mg, sg, n_rows):
    m8 = mg[0].reshape(_PACK, 2)
    s8 = sg[0].reshape(_PACK, 2)
    m2 = jnp.max(m8, axis=0)
    s2 = jnp.sum(s8 * jnp.exp(m8 - m2[None, :]), axis=0)
    g, _, b = z_t.shape
    z5 = z_t.reshape(g, _PACK, 2, b)
    ex = (jnp.exp(z5 - m2[None, None, :, None])
          * (1.0 / s2)[None, None, :, None])
    return jnp.transpose(ex, (0, 3, 1, 2)).reshape(n_rows, 2)


def kernel(x, edge_index, edge_attr, u, params):
    n_nodes, n_edges = x.shape[0], edge_attr.shape[0]
    src = edge_index[0].astype(jnp.int32)
    dst = edge_index[1].astype(jnp.int32)
    f32 = jnp.float32
    r = lambda v: jnp.reshape(v, (1, -1))
    eye8 = jnp.eye(_PACK, dtype=f32)
    bd = lambda w: jnp.kron(eye8, w)
    t8 = lambda v: jnp.tile(jnp.reshape(v, (1, -1)), (1, _PACK))
    mavg = jnp.kron(eye8, jnp.full((H, H), 1.0 / H, f32))

    pe, pn, pg = params["enc_e"], params["enc_n"], params["enc_g"]
    ce, cn = params["core_e"], params["core_n"]
    oe, on = params["out_e"], params["out_n"]
    w1, w2, w3, w4 = (ce["W"][0:16], ce["W"][16:32], ce["W"][32:48],
                      ce["W"][48:64])
    wnn, wna = cn["W"][0:16], cn["W"][16:32]

    xr = jnp.reshape(x, (n_nodes // _PACK, _PACK * x.shape[1]))
    n_p, p2_p, p3_p, c_t = _node_encode(
        xr, u, bd(pn["W"]), t8(pn["b"]), t8(pn["g"]), t8(pn["bt"]), mavg,
        bd(w2), bd(w3),
        pg["W"], r(pg["b"]), r(pg["g"]), r(pg["bt"]),
        jnp.tile(w4, (1, _PACK)), t8(ce["b"]))

    p2 = jnp.reshape(p2_p, (n_nodes, H))
    p3 = jnp.reshape(p3_p, (n_nodes, H))
    ea_t = jnp.swapaxes(edge_attr, 0, 1)
    gs, gd, ea_lin = _make_gather(n_edges)(p2, p3, src, dst, ea_t)

    ea_p = jnp.reshape(ea_lin, (n_edges // _PACK, _LANES))
    gs_p = jnp.reshape(gs, (n_edges // _PACK, _LANES))
    gd_p = jnp.reshape(gd, (n_edges // _PACK, _LANES))
    e2_p, ze_t, mg_e, sg_e = _edge_core(
        ea_p, gs_p, gd_p, c_t, bd(pe["W"]), t8(pe["b"]), t8(pe["g"]),
        t8(pe["bt"]), mavg, bd(w1), t8(ce["g"]), t8(ce["bt"]),
        bd(oe["W"]), t8(oe["b"]))

    edge_out = _finish_softmax(ze_t, mg_e, sg_e, n_edges)

    e2 = jnp.reshape(e2_p, (n_edges, H))
    aggp = _make_scatter(n_edges, n_nodes)(e2, dst)
    agg_p = jnp.reshape(aggp, (2, n_nodes // _PACK, _LANES))

    zn_t, mg_n, sg_n = _node_core(
        n_p, agg_p[0], agg_p[1], bd(wnn), bd(wna), t8(cn["b"]), t8(cn["g"]),
        t8(cn["bt"]), mavg, bd(on["W"]), t8(on["b"]))

    node_out = _finish_softmax(zn_t[None], mg_n, sg_n, n_nodes)

    glob_out = jnp.ones((1, 1), f32)
    return edge_out, node_out, glob_out


# speedup vs baseline: 7.8504x; 1.0764x over previous
import functools

import jax
import jax.numpy as jnp
from jax import lax
from jax.experimental import pallas as pl
from jax.experimental.pallas import tpu as pltpu
from jax.experimental.pallas import tpu_sc as plsc

H = 16
_PACK = 8
_LANES = _PACK * H
_EDGE_PBLOCK = 5000


def _ln_relu_packed(h, m_ref, g_t, bt_t):
    mavg = m_ref[...]
    mu = h @ mavg
    d = h - mu
    var = (d * d) @ mavg
    return jax.nn.relu(d * lax.rsqrt(var + 1e-5) * g_t + bt_t)


def _node_encode_body(xr_ref, u_ref, wnbd_ref, bn_ref, gn_ref, btn_ref,
                      mavg_ref, w2bd_ref, w3bd_ref,
                      wg_ref, bg_ref, gg_ref, btg_ref, w4t_ref, bce_ref,
                      n_ref, p2_ref, p3_ref, c_ref):
    h = xr_ref[...] @ wnbd_ref[...] + bn_ref[...]
    n = _ln_relu_packed(h, mavg_ref, gn_ref[...], btn_ref[...])
    n_ref[...] = n
    p2_ref[...] = n @ w2bd_ref[...]
    p3_ref[...] = n @ w3bd_ref[...]
    hg = u_ref[...] @ wg_ref[...] + bg_ref[...]
    mu = jnp.mean(hg, axis=-1, keepdims=True)
    var = jnp.mean((hg - mu) ** 2, axis=-1, keepdims=True)
    g = jax.nn.relu((hg - mu) * lax.rsqrt(var + 1e-5) * gg_ref[...]
                    + btg_ref[...])
    c_ref[...] = g @ w4t_ref[...] + bce_ref[...]


def _node_encode(xr, u, wnbd, bn, gn, btn, mavg, w2bd, w3bd,
                 wg, bg, gg, btg, w4t, bce, interpret=False):
    rows = xr.shape[0]
    out_shape = [
        jax.ShapeDtypeStruct((rows, _LANES), jnp.float32),
        jax.ShapeDtypeStruct((rows, _LANES), jnp.float32),
        jax.ShapeDtypeStruct((rows, _LANES), jnp.float32),
        jax.ShapeDtypeStruct((1, _LANES), jnp.float32),
    ]
    return pl.pallas_call(_node_encode_body, out_shape=out_shape,
                          interpret=interpret)(
        xr, u, wnbd, bn, gn, btn, mavg, w2bd, w3bd,
        wg, bg, gg, btg, w4t, bce)


@functools.cache
def _make_gather(n_edges):
    nw = 32
    per = n_edges // nw
    ch = 1000
    nch = per // ch
    mesh = plsc.VectorSubcoreMesh(core_axis_name="c", subcore_axis_name="s")

    @functools.partial(
        pl.kernel, mesh=mesh,
        compiler_params=pltpu.CompilerParams(use_tc_tiling_on_sc=False,
                                             needs_layout_passes=False),
        out_type=[jax.ShapeDtypeStruct((n_edges, H), jnp.float32),
                  jax.ShapeDtypeStruct((n_edges, H), jnp.float32),
                  jax.ShapeDtypeStruct((n_edges, H), jnp.float32)],
        scratch_types=[pltpu.VMEM((2, ch), jnp.int32),
                       pltpu.VMEM((2, ch), jnp.int32),
                       pltpu.VMEM((2, ch, H), jnp.float32),
                       pltpu.VMEM((2, ch, H), jnp.float32),
                       pltpu.VMEM((2, H, ch), jnp.float32),
                       pltpu.VMEM((ch, H), jnp.float32),
                       pltpu.SemaphoreType.DMA,
                       pltpu.SemaphoreType.DMA,
                       pltpu.SemaphoreType.DMA,
                       pltpu.SemaphoreType.DMA,
                       pltpu.SemaphoreType.DMA,
                       pltpu.SemaphoreType.DMA],
    )
    def gather(p2_hbm, p3_hbm, src_hbm, dst_hbm, eat_hbm,
               gs_hbm, gd_hbm, ea_out_hbm,
               idxa, idxb, rowsa, rowsb, sbuf, rowse,
               sa0, sa1, sb0, sb1, se0, se1):
        wid = lax.axis_index("s") * 2 + lax.axis_index("c")
        base = wid * per
        lanes = lax.iota(jnp.int32, 16)
        sems = ((sa0, sb0, se0), (sa1, sb1, se1))

        def start(c, p):
            off = base + c * ch
            sa, sb, se = sems[p]
            pltpu.sync_copy(src_hbm.at[pl.ds(off, ch)], idxa.at[p])
            pltpu.sync_copy(dst_hbm.at[pl.ds(off, ch)], idxb.at[p])
            pltpu.async_copy(p2_hbm.at[idxa.at[p]], rowsa.at[p], sa)
            pltpu.async_copy(p3_hbm.at[idxb.at[p]], rowsb.at[p], sb)
            pltpu.async_copy(eat_hbm.at[:, pl.ds(off, ch)], sbuf.at[p], se)

        def finish(c, p):
            off = base + c * ch
            sa, sb, se = sems[p]
            pltpu.make_async_copy(eat_hbm.at[:, pl.ds(off, ch)],
                                  sbuf.at[p], se).wait()

            def trans8(e8, carry2):
                e = e8 * 8
                for t in range(8):
                    col = jnp.full((16,), e + t, jnp.int32)
                    rowse[e + t, :] = plsc.load_gather(sbuf.at[p],
                                                       [lanes, col])
                return carry2

            lax.fori_loop(0, ch // 8, trans8, 0)
            pltpu.sync_copy(rowse, ea_out_hbm.at[pl.ds(off, ch)])
            pltpu.make_async_copy(p2_hbm.at[idxa.at[p]],
                                  rowsa.at[p], sa).wait()
            pltpu.sync_copy(rowsa.at[p], gs_hbm.at[pl.ds(off, ch)])
            pltpu.make_async_copy(p3_hbm.at[idxb.at[p]],
                                  rowsb.at[p], sb).wait()
            pltpu.sync_copy(rowsb.at[p], gd_hbm.at[pl.ds(off, ch)])

        start(0, 0)

        def body(c2, carry):
            c = c2 * 2
            start(c + 1, 1)
            finish(c, 0)

            @pl.when(c2 < nch // 2 - 1)
            def _():
                start(c + 2, 0)

            finish(c + 1, 1)
            return carry

        lax.fori_loop(0, nch // 2, body, 0)

    return gather


@functools.cache
def _make_scatter(n_edges, n_nodes):
    nw = 32
    per = n_edges // nw
    ch = 2000
    nch = per // ch
    zrows = n_nodes // 16
    mesh = plsc.VectorSubcoreMesh(core_axis_name="c", subcore_axis_name="s")

    @functools.partial(
        pl.kernel, mesh=mesh,
        compiler_params=pltpu.CompilerParams(use_tc_tiling_on_sc=False),
        out_type=jax.ShapeDtypeStruct((2, n_nodes, H), jnp.float32),
        scratch_types=[pltpu.VMEM((ch,), jnp.int32),
                       pltpu.VMEM((ch, H), jnp.float32),
                       pltpu.VMEM((zrows, H), jnp.float32),
                       pltpu.VMEM_SHARED((n_nodes, H), jnp.float32),
                       pltpu.SemaphoreType.DMA],
    )
    def scatter(e2_hbm, dst_hbm, out_hbm, idx, rows, tbuf, acc_sh, sem):
        cid = lax.axis_index("c")
        sid = lax.axis_index("s")

        def zero_row(r, carry):
            tbuf[r, :] = jnp.zeros((H,), jnp.float32)
            return carry

        lax.fori_loop(0, zrows, zero_row, 0)
        pltpu.sync_copy(tbuf, acc_sh.at[pl.ds(sid * zrows, zrows)])
        plsc.subcore_barrier()

        wid = sid * 2 + cid
        base = wid * per

        def body(c, carry):
            off = base + c * ch
            pltpu.sync_copy(dst_hbm.at[pl.ds(off, ch)], idx)
            cp = pltpu.async_copy(e2_hbm.at[pl.ds(off, ch)], rows, sem)
            cp.wait()
            pltpu.sync_copy(rows, acc_sh.at[idx], add=True)
            return carry

        lax.fori_loop(0, nch, body, 0)
        plsc.subcore_barrier()
        pltpu.sync_copy(acc_sh.at[pl.ds(sid * zrows, zrows)], tbuf)
        pltpu.sync_copy(tbuf, out_hbm.at[cid].at[pl.ds(sid * zrows, zrows)])

    return scatter


def _edge_core_body(ea_ref, gs_ref, gd_ref, c_ref,
                    webd_ref, be_ref, ge_ref, bte_ref, mavg_ref,
                    w1bd_ref, gc_ref, btc_ref, woebd_ref, boe_ref,
                    e2_ref, ze_ref, m_ref, s_ref):
    i = pl.program_id(0)
    e_enc = _ln_relu_packed(ea_ref[...] @ webd_ref[...] + be_ref[...],
                            mavg_ref, ge_ref[...], bte_ref[...])
    h = e_enc @ w1bd_ref[...] + gs_ref[...] + gd_ref[...] + c_ref[...]
    e2 = _ln_relu_packed(h, mavg_ref, gc_ref[...], btc_ref[...])
    e2_ref[...] = e2
    ze = e2 @ woebd_ref[...] + boe_ref[...]
    ze_ref[...] = jnp.transpose(ze)[None]

    @pl.when(i == 0)
    def _():
        m_ref[...] = jnp.full((1, H), -jnp.inf, jnp.float32)
        s_ref[...] = jnp.zeros((1, H), jnp.float32)

    bm = jnp.max(ze, axis=0, keepdims=True)
    m_old = m_ref[...]
    m_new = jnp.maximum(m_old, bm)
    s_ref[...] = (s_ref[...] * jnp.exp(m_old - m_new)
                  + jnp.sum(jnp.exp(ze - m_new), axis=0, keepdims=True))
    m_ref[...] = m_new


def _edge_core(ea, gs, gd, c, webd, be, ge, bte, mavg, w1bd, gc, btc,
               woebd, boe, interpret=False):
    rows = ea.shape[0]
    blk = _EDGE_PBLOCK
    grid = rows // blk
    row_spec = pl.BlockSpec((blk, _LANES), lambda i: (i, 0))
    full = lambda a: pl.BlockSpec(a.shape, lambda i: tuple(0 for _ in a.shape))
    out_shape = [
        jax.ShapeDtypeStruct((rows, _LANES), jnp.float32),
        jax.ShapeDtypeStruct((grid, H, blk), jnp.float32),
        jax.ShapeDtypeStruct((1, H), jnp.float32),
        jax.ShapeDtypeStruct((1, H), jnp.float32),
    ]
    return pl.pallas_call(
        _edge_core_body,
        grid=(grid,),
        in_specs=[row_spec, row_spec, row_spec, full(c), full(webd), full(be),
                  full(ge), full(bte), full(mavg), full(w1bd), full(gc),
                  full(btc), full(woebd), full(boe)],
        out_specs=[row_spec, pl.BlockSpec((1, H, blk), lambda i: (i, 0, 0)),
                   pl.BlockSpec((1, H), lambda i: (0, 0)),
                   pl.BlockSpec((1, H), lambda i: (0, 0))],
        out_shape=out_shape,
        interpret=interpret,
    )(ea, gs, gd, c, webd, be, ge, bte, mavg, w1bd, gc, btc, woebd, boe)


def _node_core_body(n_ref, a0_ref, a1_ref, wnnbd_ref, wnabd_ref, bcn_ref,
                    gcn_ref, btcn_ref, mavg_ref, wonbd_ref, bon_ref,
                    zn_ref, m_ref, s_ref):
    agg = a0_ref[...] + a1_ref[...]
    h = n_ref[...] @ wnnbd_ref[...] + agg @ wnabd_ref[...] + bcn_ref[...]
    n2 = _ln_relu_packed(h, mavg_ref, gcn_ref[...], btcn_ref[...])
    zn = n2 @ wonbd_ref[...] + bon_ref[...]
    zn_ref[...] = jnp.transpose(zn)
    m = jnp.max(zn, axis=0, keepdims=True)
    m_ref[...] = m
    s_ref[...] = jnp.sum(jnp.exp(zn - m), axis=0, keepdims=True)


def _node_core(n, a0, a1, wnnbd, wnabd, bcn, gcn, btcn, mavg, wonbd, bon,
               interpret=False):
    rows = n.shape[0]
    out_shape = [
        jax.ShapeDtypeStruct((H, rows), jnp.float32),
        jax.ShapeDtypeStruct((1, H), jnp.float32),
        jax.ShapeDtypeStruct((1, H), jnp.float32),
    ]
    return pl.pallas_call(_node_core_body, out_shape=out_shape,
                          interpret=interpret)(
        n, a0, a1, wnnbd, wnabd, bcn, gcn, btcn, mavg, wonbd, bon)


def _finish_softmax(z_t, mg, sg, n_rows):
    m8 = mg[0].reshape(_PACK, 2)
    s8 = sg[0].reshape(_PACK, 2)
    m2 = jnp.max(m8, axis=0)
    s2 = jnp.sum(s8 * jnp.exp(m8 - m2[None, :]), axis=0)
    g, _, b = z_t.shape
    z5 = z_t.reshape(g, _PACK, 2, b)
    ex = (jnp.exp(z5 - m2[None, None, :, None])
          * (1.0 / s2)[None, None, :, None])
    return jnp.transpose(ex, (0, 3, 1, 2)).reshape(n_rows, 2)


def kernel(x, edge_index, edge_attr, u, params):
    n_nodes, n_edges = x.shape[0], edge_attr.shape[0]
    src = edge_index[0].astype(jnp.int32)
    dst = edge_index[1].astype(jnp.int32)
    f32 = jnp.float32
    r = lambda v: jnp.reshape(v, (1, -1))
    eye8 = jnp.eye(_PACK, dtype=f32)
    bd = lambda w: jnp.kron(eye8, w)
    t8 = lambda v: jnp.tile(jnp.reshape(v, (1, -1)), (1, _PACK))
    mavg = jnp.kron(eye8, jnp.full((H, H), 1.0 / H, f32))

    pe, pn, pg = params["enc_e"], params["enc_n"], params["enc_g"]
    ce, cn = params["core_e"], params["core_n"]
    oe, on = params["out_e"], params["out_n"]
    w1, w2, w3, w4 = (ce["W"][0:16], ce["W"][16:32], ce["W"][32:48],
                      ce["W"][48:64])
    wnn, wna = cn["W"][0:16], cn["W"][16:32]

    xr = jnp.reshape(x, (n_nodes // _PACK, _PACK * x.shape[1]))
    n_p, p2_p, p3_p, c_t = _node_encode(
        xr, u, bd(pn["W"]), t8(pn["b"]), t8(pn["g"]), t8(pn["bt"]), mavg,
        bd(w2), bd(w3),
        pg["W"], r(pg["b"]), r(pg["g"]), r(pg["bt"]),
        jnp.tile(w4, (1, _PACK)), t8(ce["b"]))

    p2 = jnp.reshape(p2_p, (n_nodes, H))
    p3 = jnp.reshape(p3_p, (n_nodes, H))
    ea_t = jnp.swapaxes(edge_attr, 0, 1)
    gs, gd, ea_lin = _make_gather(n_edges)(p2, p3, src, dst, ea_t)

    ea_p = jnp.reshape(ea_lin, (n_edges // _PACK, _LANES))
    gs_p = jnp.reshape(gs, (n_edges // _PACK, _LANES))
    gd_p = jnp.reshape(gd, (n_edges // _PACK, _LANES))
    e2_p, ze_t, mg_e, sg_e = _edge_core(
        ea_p, gs_p, gd_p, c_t, bd(pe["W"]), t8(pe["b"]), t8(pe["g"]),
        t8(pe["bt"]), mavg, bd(w1), t8(ce["g"]), t8(ce["bt"]),
        bd(oe["W"]), t8(oe["b"]))

    edge_out = _finish_softmax(ze_t, mg_e, sg_e, n_edges)

    e2 = jnp.reshape(e2_p, (n_edges, H))
    aggp = _make_scatter(n_edges, n_nodes)(e2, dst)
    agg_p = jnp.reshape(aggp, (2, n_nodes // _PACK, _LANES))

    zn_t, mg_n, sg_n = _node_core(
        n_p, agg_p[0], agg_p[1], bd(wnn), bd(wna), t8(cn["b"]), t8(cn["g"]),
        t8(cn["bt"]), mavg, bd(on["W"]), t8(on["b"]))

    node_out = _finish_softmax(zn_t[None], mg_n, sg_n, n_nodes)

    glob_out = jnp.ones((1, 1), f32)
    return edge_out, node_out, glob_out
